# baseline (speedup 1.0000x reference)
import functools

import jax
import jax.numpy as jnp
from jax import lax
from jax.experimental import pallas as pl
from jax.experimental.pallas import tpu as pltpu
from jax.experimental.pallas import tpu_sc as plsc


def _sc_pos_encode(row_embed, col_embed, batch, h, w, nf):
    info = plsc.get_sparse_core_info()
    nc, ns = info.num_cores, info.num_subcores
    nw = nc * ns
    nt = -(-h // nw)
    c2 = 2 * nf
    dt = row_embed.dtype
    mesh = plsc.VectorSubcoreMesh(core_axis_name="c", subcore_axis_name="s")

    @functools.partial(
        pl.kernel,
        mesh=mesh,
        out_type=jax.ShapeDtypeStruct((batch, h, w, c2), dt),
        scratch_types=[
            pltpu.VMEM((2, w, c2), dt),
            pltpu.VMEM((1, nf), dt),
            pltpu.SemaphoreType.DMA((2,)),
        ],
    )
    def k(row_hbm, col_hbm, out_hbm, slab, rowbuf, sem):
        wid = lax.axis_index("s") * nc + lax.axis_index("c")
        for s in range(2):
            pltpu.sync_copy(col_hbm, slab.at[s, :, pl.ds(0, nf)])
        for t in range(nt):
            y = wid + t * nw
            s = t % 2

            @pl.when(y < h)
            def _(y=y, s=s):
                pltpu.sync_copy(row_hbm.at[pl.ds(y, 1), :], rowbuf)
                vecs = [rowbuf.at[0][pl.ds(16 * i, 16)] for i in range(nf // 16)]

                def body(x, carry):
                    r = slab.at[s, x]
                    for i in range(nf // 16):
                        r[pl.ds(nf + 16 * i, 16)] = vecs[i]
                    return carry

                lax.fori_loop(0, w, body, 0)
                cps = [
                    pltpu.async_copy(slab.at[s], out_hbm.at[b, y], sem.at[s])
                    for b in range(batch)
                ]
                for c in cps:
                    c.wait()

    return k(row_embed, col_embed)


def kernel(mask, row_embed, col_embed):
    batch = mask.shape[0]
    h, w = mask.shape[-2], mask.shape[-1]
    nf = row_embed.shape[1]
    p = _sc_pos_encode(row_embed, col_embed, batch, h, w, nf)
    return jnp.transpose(p, (0, 3, 1, 2))


# speedup vs baseline: 1.4720x; 1.4720x over previous
import jax
import jax.numpy as jnp
from jax.experimental import pallas as pl

_YBLK = 40
_BBLK = 1


def _bcast_body(row_ref, col_ref, out_ref):
    nf = row_ref.shape[1]
    re = row_ref[...]
    ce = col_ref[...]
    yb, w = out_ref.shape[1], out_ref.shape[2]
    nb = out_ref.shape[0]
    out_ref[:, :, :, nf:] = jnp.broadcast_to(re[None, :, None, :], (nb, yb, w, nf))
    out_ref[:, :, :, :nf] = jnp.broadcast_to(ce[None, None, :, :], (nb, yb, w, nf))


def kernel(mask, row_embed, col_embed):
    batch = mask.shape[0]
    h, w = mask.shape[-2], mask.shape[-1]
    nf = row_embed.shape[1]

    grid = (batch // _BBLK, h // _YBLK)
    p = pl.pallas_call(
        _bcast_body,
        grid=grid,
        in_specs=[
            pl.BlockSpec((_YBLK, nf), lambda b, i: (i, 0)),
            pl.BlockSpec((w, nf), lambda b, i: (0, 0)),
        ],
        out_specs=pl.BlockSpec((_BBLK, _YBLK, w, 2 * nf), lambda b, i: (b, i, 0, 0)),
        out_shape=jax.ShapeDtypeStruct((batch, h, w, 2 * nf), row_embed.dtype),
    )(row_embed, col_embed)
    return jnp.transpose(p, (0, 3, 1, 2))
